# Initial kernel scaffold; baseline (speedup 1.0000x reference)
import jax
import jax.numpy as jnp
from jax.experimental import pallas as pl


def kernel(h, edge_attr, edge_index, W_src, att_src, att_dst, W_edge, att_edge, bias_gat, W_lin, b_lin, ln_gamma, ln_beta):
    raise NotImplementedError("write your pallas kernel here")


# speedup vs baseline: 1.1818x; 1.1818x over previous
import functools
import jax
import jax.numpy as jnp
from jax import lax
from jax.experimental import pallas as pl

N = 10000
E = 160000
D = 256
H = 4

NB_N = 1000
NB_E = 2000


def _k1a_body(h_ref, w_ref, o_ref):
    o_ref[0] = jnp.dot(h_ref[...], w_ref[0],
                       preferred_element_type=jnp.float32)


def _k1b_body(h_ref, v_ref, o_ref):
    o_ref[...] = jnp.dot(h_ref[...], v_ref[...],
                         preferred_element_type=jnp.float32)


def _k2_body(ea_ref, v_ref, o_ref):
    r = jnp.dot(ea_ref[...], v_ref[...], preferred_element_type=jnp.float32)
    col = lax.broadcasted_iota(jnp.int32, r.shape, 1)
    o_ref[...] = jnp.where(col == 4, 1.0, r)


def _k4_body(wsum_ref, x4_ref, t_ref, dacc_ref, aes_ref, wlin_ref,
             bgat_ref, blin_ref, lng_ref, lnb_ref, o_ref):
    asrc = t_ref[:, 0:4]
    adst = t_ref[:, 4:8]
    cnt = aes_ref[:, 4:5]
    loop_ae = aes_ref[:, 0:4] / jnp.maximum(cnt, 1.0)
    al = asrc + adst + loop_ae
    al = jnp.where(al > 0, al, 0.2 * al)
    wl = jnp.exp(al)
    denom = dacc_ref[:, 0:4] + wl + 1e-16
    cols = []
    for hh in range(H):
        numer_h = wsum_ref[:, hh, :] + wl[:, hh:hh + 1] * x4_ref[hh]
        cols.append(numer_h / denom[:, hh:hh + 1])
    z = jnp.concatenate(cols, axis=1) + bgat_ref[...]
    y = jnp.dot(z, wlin_ref[...], preferred_element_type=jnp.float32)
    y = y + blin_ref[...]
    mu = jnp.mean(y, axis=-1, keepdims=True)
    var = jnp.mean((y - mu) ** 2, axis=-1, keepdims=True)
    o_ref[...] = (y - mu) * lax.rsqrt(var + 1e-5) * lng_ref[...] + lnb_ref[...]


def kernel(h, edge_attr, edge_index, W_src, att_src, att_dst, W_edge,
           att_edge, bias_gat, W_lin, b_lin, ln_gamma, ln_beta):
    f32 = jnp.float32
    Ws3 = W_src.reshape(D, H, D)
    Vs = (Ws3 * att_src).sum(-1)
    Vd = (Ws3 * att_dst).sum(-1)
    Ve = (W_edge.reshape(D, H, D) * att_edge).sum(-1)
    Vcat = jnp.concatenate([Vs, Vd], axis=1)
    Ve8 = jnp.pad(Ve, ((0, 0), (0, 4)))

    src, dst = edge_index[0], edge_index[1]

    x4 = pl.pallas_call(
        _k1a_body,
        grid=(H, N // NB_N),
        in_specs=[
            pl.BlockSpec((NB_N, D), lambda hi, ni: (ni, 0)),
            pl.BlockSpec((1, D, D), lambda hi, ni: (hi, 0, 0)),
        ],
        out_specs=pl.BlockSpec((1, NB_N, D), lambda hi, ni: (hi, ni, 0)),
        out_shape=jax.ShapeDtypeStruct((H, N, D), f32),
    )(h, Ws3.transpose(1, 0, 2))

    T = pl.pallas_call(
        _k1b_body,
        grid=(N // NB_N,),
        in_specs=[
            pl.BlockSpec((NB_N, D), lambda ni: (ni, 0)),
            pl.BlockSpec((D, 8), lambda ni: (0, 0)),
        ],
        out_specs=pl.BlockSpec((NB_N, 8), lambda ni: (ni, 0)),
        out_shape=jax.ShapeDtypeStruct((N, 8), f32),
    )(h, Vcat)

    ae8 = pl.pallas_call(
        _k2_body,
        grid=(E // NB_E,),
        in_specs=[
            pl.BlockSpec((NB_E, D), lambda ei: (ei, 0)),
            pl.BlockSpec((D, 8), lambda ei: (0, 0)),
        ],
        out_specs=pl.BlockSpec((NB_E, 8), lambda ei: (ei, 0)),
        out_shape=jax.ShapeDtypeStruct((E, 8), f32),
    )(edge_attr, Ve8)

    alpha8 = T[src] * jnp.array([1.0] * 4 + [0.0] * 4, f32) \
        + T[dst][:, [4, 5, 6, 7, 0, 1, 2, 3]] * jnp.array([1.0] * 4 + [0.0] * 4, f32) \
        + ae8
    alpha8 = jnp.where(alpha8 > 0, alpha8, 0.2 * alpha8)
    w8 = jnp.exp(alpha8)
    dacc = jax.ops.segment_sum(w8, dst, num_segments=N)
    aes8 = jax.ops.segment_sum(ae8, dst, num_segments=N)

    msg = w8[:, :4, None] * jnp.take(x4, src, axis=1).transpose(1, 0, 2)
    wsum = jax.ops.segment_sum(msg, dst, num_segments=N)

    out = pl.pallas_call(
        _k4_body,
        grid=(N // NB_N,),
        in_specs=[
            pl.BlockSpec((NB_N, H, D), lambda ni: (ni, 0, 0)),
            pl.BlockSpec((H, NB_N, D), lambda ni: (0, ni, 0)),
            pl.BlockSpec((NB_N, 8), lambda ni: (ni, 0)),
            pl.BlockSpec((NB_N, 8), lambda ni: (ni, 0)),
            pl.BlockSpec((NB_N, 8), lambda ni: (ni, 0)),
            pl.BlockSpec((H * D, D), lambda ni: (0, 0)),
            pl.BlockSpec((1, H * D), lambda ni: (0, 0)),
            pl.BlockSpec((1, D), lambda ni: (0, 0)),
            pl.BlockSpec((1, D), lambda ni: (0, 0)),
            pl.BlockSpec((1, D), lambda ni: (0, 0)),
        ],
        out_specs=pl.BlockSpec((NB_N, D), lambda ni: (ni, 0)),
        out_shape=jax.ShapeDtypeStruct((N, D), f32),
    )(wsum, x4, T, dacc, aes8, W_lin,
      bias_gat.reshape(1, H * D), b_lin.reshape(1, D),
      ln_gamma.reshape(1, D), ln_beta.reshape(1, D))
    return out


# speedup vs baseline: 1.2235x; 1.0353x over previous
import functools
import jax
import jax.numpy as jnp
from jax import lax
from jax.experimental import pallas as pl
from jax.experimental.pallas import tpu as pltpu
from jax.experimental.pallas import tpu_sc as plsc

N = 10000
E = 160000
D = 256
H = 4

NB_N = 1000
NB_E = 2048
EP = 163840
NW = 32
SUB = 64
NSUB = 80
NP = 10112
NPT = NP // 16


def _k1a_body(h_ref, w_ref, o_ref):
    o_ref[0] = jnp.dot(h_ref[...], w_ref[0],
                       preferred_element_type=jnp.float32)


def _k1b_body(h_ref, v_ref, o_ref):
    o_ref[...] = jnp.dot(h_ref[...], v_ref[...],
                         preferred_element_type=jnp.float32)


def _k2_body(ea_ref, v_ref, o_ref):
    r = jnp.dot(ea_ref[...], v_ref[...], preferred_element_type=jnp.float32)
    col = lax.broadcasted_iota(jnp.int32, r.shape, 1)
    o_ref[...] = jnp.where(col == 4, 1.0, r)


def _k4_body(wsum_ref, x4_ref, tc_ref, parts_ref, wlin_ref,
             bgat_ref, blin_ref, lng_ref, lnb_ref, o_ref):
    asrc = tc_ref[:, 0:4]
    adst = tc_ref[:, 16:20]
    acc = parts_ref[0] + parts_ref[1]
    cnt = acc[:, 20:21]
    loop_ae = acc[:, 16:20] / jnp.maximum(cnt, 1.0)
    al = asrc + adst + loop_ae
    al = jnp.where(al > 0, al, 0.2 * al)
    wl = jnp.exp(al)
    denom = acc[:, 0:4] + wl + 1e-16
    cols = []
    for hh in range(H):
        numer_h = wsum_ref[:, hh, :] + wl[:, hh:hh + 1] * x4_ref[hh]
        cols.append(numer_h / denom[:, hh:hh + 1])
    z = jnp.concatenate(cols, axis=1) + bgat_ref[...]
    y = jnp.dot(z, wlin_ref[...], preferred_element_type=jnp.float32)
    y = y + blin_ref[...]
    mu = jnp.mean(y, axis=-1, keepdims=True)
    var = jnp.mean((y - mu) ** 2, axis=-1, keepdims=True)
    o_ref[...] = (y - mu) * lax.rsqrt(var + 1e-5) * lng_ref[...] + lnb_ref[...]


def _s1_body(src1d, dst1d, tc_hbm, ae16f, zrows128, w16_out, acc_out,
             src_i, dst_i, ts_v, td_v, ae_v, w_v, wa_v, acc_sh):
    cid = lax.axis_index("c")
    sid = lax.axis_index("s")
    wid = sid * 2 + cid
    @pl.when(sid == 0)
    def _init():
        pltpu.sync_copy(zrows128, acc_sh)
    pltpu.sync_copy(zrows128.at[pl.ds(0, SUB)], wa_v)
    plsc.subcore_barrier()

    def sub_body(s, carry):
        e0 = (wid * NSUB + s) * SUB
        pltpu.sync_copy(src1d.at[pl.ds(e0, SUB)], src_i)
        pltpu.sync_copy(dst1d.at[pl.ds(e0, SUB)], dst_i)
        pltpu.sync_copy(ae16f.at[pl.ds(e0 * 16, SUB * 16)], ae_v)
        pltpu.sync_copy(tc_hbm.at[src_i], ts_v)
        pltpu.sync_copy(tc_hbm.at[dst_i], td_v)

        def cmp(e, c2):
            ar = ae_v[pl.ds(e * 16, 16)]
            al = ts_v[e, 0:16] + td_v[e, 16:32] + ar
            al = jnp.where(al > 0, al, 0.2 * al)
            w = jnp.exp(al)
            w_v[pl.ds(e * 16, 16)] = w
            wa_v[e, 0:16] = w
            wa_v[e, 16:32] = ar
            return c2
        lax.fori_loop(0, SUB, cmp, 0)

        pltpu.sync_copy(wa_v, acc_sh.at[dst_i], add=True)
        pltpu.sync_copy(w_v, w16_out.at[pl.ds(e0 * 16, SUB * 16)])
        return carry

    lax.fori_loop(0, NSUB, sub_body, 0)
    plsc.subcore_barrier()
    @pl.when(sid == 0)
    def _writeout():
        pltpu.sync_copy(acc_sh, acc_out.at[cid])


def _run_s1(src_p, dst_p, tc_pad, ae16f, zrows128):
    f32 = jnp.float32
    mesh = plsc.VectorSubcoreMesh(core_axis_name="c", subcore_axis_name="s")
    fn = pl.kernel(
        _s1_body,
        mesh=mesh,
        out_type=[
            jax.ShapeDtypeStruct((EP * 16,), f32),
            jax.ShapeDtypeStruct((2, NP, 128), f32),
        ],
        scratch_types=[
            pltpu.VMEM((SUB,), jnp.int32),
            pltpu.VMEM((SUB,), jnp.int32),
            pltpu.VMEM((SUB, 128), f32),
            pltpu.VMEM((SUB, 128), f32),
            pltpu.VMEM((SUB * 16,), f32),
            pltpu.VMEM((SUB * 16,), f32),
            pltpu.VMEM((SUB, 128), f32),
            pltpu.VMEM_SHARED((NP, 128), f32),
        ],
    )
    return fn(src_p, dst_p, tc_pad, ae16f, zrows128)


def kernel(h, edge_attr, edge_index, W_src, att_src, att_dst, W_edge,
           att_edge, bias_gat, W_lin, b_lin, ln_gamma, ln_beta):
    f32 = jnp.float32
    Ws3 = W_src.reshape(D, H, D)
    Vs = (Ws3 * att_src).sum(-1)
    Vd = (Ws3 * att_dst).sum(-1)
    Ve = (W_edge.reshape(D, H, D) * att_edge).sum(-1)
    V128 = jnp.zeros((D, 128), f32)
    V128 = V128.at[:, 0:4].set(Vs).at[:, 16:20].set(Vd)
    Ve16 = jnp.pad(Ve, ((0, 0), (0, 12)))

    src, dst = edge_index[0], edge_index[1]

    x4 = pl.pallas_call(
        _k1a_body,
        grid=(H, N // NB_N),
        in_specs=[
            pl.BlockSpec((NB_N, D), lambda hi, ni: (ni, 0)),
            pl.BlockSpec((1, D, D), lambda hi, ni: (hi, 0, 0)),
        ],
        out_specs=pl.BlockSpec((1, NB_N, D), lambda hi, ni: (hi, ni, 0)),
        out_shape=jax.ShapeDtypeStruct((H, N, D), f32),
    )(h, Ws3.transpose(1, 0, 2))

    Tc = pl.pallas_call(
        _k1b_body,
        grid=(N // NB_N,),
        in_specs=[
            pl.BlockSpec((NB_N, D), lambda ni: (ni, 0)),
            pl.BlockSpec((D, 128), lambda ni: (0, 0)),
        ],
        out_specs=pl.BlockSpec((NB_N, 128), lambda ni: (ni, 0)),
        out_shape=jax.ShapeDtypeStruct((N, 128), f32),
    )(h, V128)

    ea_p = jnp.pad(edge_attr, ((0, EP - E), (0, 0)))
    ae16p = pl.pallas_call(
        _k2_body,
        grid=(EP // NB_E,),
        in_specs=[
            pl.BlockSpec((NB_E, D), lambda ei: (ei, 0)),
            pl.BlockSpec((D, 16), lambda ei: (0, 0)),
        ],
        out_specs=pl.BlockSpec((NB_E, 16), lambda ei: (ei, 0)),
        out_shape=jax.ShapeDtypeStruct((EP, 16), f32),
    )(ea_p, Ve16)

    src_p = jnp.pad(src, (0, EP - E))
    dst_p = jnp.pad(dst, (0, EP - E), constant_values=N)
    tc_pad = jnp.pad(Tc, ((0, NP - N), (0, 0)))
    zrows128 = jnp.zeros((NP, 128), f32)
    w16f, parts = _run_s1(src_p, dst_p, tc_pad,
                          ae16p.reshape(EP * 16), zrows128)
    w16 = w16f.reshape(EP, 16)

    msg = w16[:E, :4, None] * jnp.take(x4, src, axis=1).transpose(1, 0, 2)
    wsum = jax.ops.segment_sum(msg, dst, num_segments=N)

    out = pl.pallas_call(
        _k4_body,
        grid=(N // NB_N,),
        in_specs=[
            pl.BlockSpec((NB_N, H, D), lambda ni: (ni, 0, 0)),
            pl.BlockSpec((H, NB_N, D), lambda ni: (0, ni, 0)),
            pl.BlockSpec((NB_N, 128), lambda ni: (ni, 0)),
            pl.BlockSpec((2, NB_N, 128), lambda ni: (0, ni, 0)),
            pl.BlockSpec((H * D, D), lambda ni: (0, 0)),
            pl.BlockSpec((1, H * D), lambda ni: (0, 0)),
            pl.BlockSpec((1, D), lambda ni: (0, 0)),
            pl.BlockSpec((1, D), lambda ni: (0, 0)),
            pl.BlockSpec((1, D), lambda ni: (0, 0)),
        ],
        out_specs=pl.BlockSpec((NB_N, D), lambda ni: (ni, 0)),
        out_shape=jax.ShapeDtypeStruct((N, D), f32),
    )(wsum, x4, Tc, parts, W_lin,
      bias_gat.reshape(1, H * D), b_lin.reshape(1, D),
      ln_gamma.reshape(1, D), ln_beta.reshape(1, D))
    return out


# speedup vs baseline: 6.1164x; 4.9992x over previous
import functools
import jax
import jax.numpy as jnp
from jax import lax
from jax.experimental import pallas as pl
from jax.experimental.pallas import tpu as pltpu
from jax.experimental.pallas import tpu_sc as plsc

N = 10000
E = 160000
D = 256
H = 4

NB_N = 1000
NB_E = 2048
EP = 163840
NW = 32
SUB = 64
NSUB = 80
NP = 10112
NPT = NP // 16
SUB3 = 128
NSUB3 = EP // (NW * SUB3)


def _k1a_body(h_ref, w_ref, o_ref):
    o_ref[0, 0] = jnp.dot(h_ref[...], w_ref[0],
                          preferred_element_type=jnp.float32)


def _k1b_body(h_ref, v_ref, o_ref):
    o_ref[...] = jnp.dot(h_ref[...], v_ref[...],
                         preferred_element_type=jnp.float32)


def _k2_body(ea_ref, v_ref, o_ref):
    r = jnp.dot(ea_ref[...], v_ref[...], preferred_element_type=jnp.float32)
    col = lax.broadcasted_iota(jnp.int32, r.shape, 1)
    o_ref[...] = jnp.where(col == 4, 1.0, r)


def _k4_body(agg_ref, x4r_ref, tc_ref, parts_ref, wlin_ref,
             bgat_ref, blin_ref, lng_ref, lnb_ref, o_ref):
    asrc = tc_ref[:, 0:4]
    adst = tc_ref[:, 16:20]
    acc = parts_ref[0] + parts_ref[1]
    cnt = acc[:, 20:21]
    loop_ae = acc[:, 16:20] / jnp.maximum(cnt, 1.0)
    al = asrc + adst + loop_ae
    al = jnp.where(al > 0, al, 0.2 * al)
    wl = jnp.exp(al)
    denom = acc[:, 0:4] + wl + 1e-16
    aggs = agg_ref[0] + agg_ref[1]
    cols = []
    for hh in range(H):
        wsum_h = jnp.concatenate([aggs[2 * hh], aggs[2 * hh + 1]], axis=1)
        x4_h = jnp.concatenate([x4r_ref[2 * hh], x4r_ref[2 * hh + 1]],
                               axis=1)
        numer_h = wsum_h + wl[:, hh:hh + 1] * x4_h
        cols.append(numer_h / denom[:, hh:hh + 1])
    z = jnp.concatenate(cols, axis=1) + bgat_ref[...]
    y = jnp.dot(z, wlin_ref[...], preferred_element_type=jnp.float32)
    y = y + blin_ref[...]
    mu = jnp.mean(y, axis=-1, keepdims=True)
    var = jnp.mean((y - mu) ** 2, axis=-1, keepdims=True)
    o_ref[...] = (y - mu) * lax.rsqrt(var + 1e-5) * lng_ref[...] + lnb_ref[...]


def _s1_body(src1d, dst1d, tc_hbm, ae16f, zrows128, w16_out, acc_out,
             src_i, dst_i, ts_v, td_v, ae_v, w_v, wa_v, acc_sh):
    cid = lax.axis_index("c")
    sid = lax.axis_index("s")
    wid = sid * 2 + cid
    @pl.when(sid == 0)
    def _init():
        pltpu.sync_copy(zrows128, acc_sh)
    pltpu.sync_copy(zrows128.at[pl.ds(0, SUB)], wa_v)
    plsc.subcore_barrier()

    def sub_body(s, carry):
        e0 = (wid * NSUB + s) * SUB
        pltpu.sync_copy(src1d.at[pl.ds(e0, SUB)], src_i)
        pltpu.sync_copy(dst1d.at[pl.ds(e0, SUB)], dst_i)
        pltpu.sync_copy(ae16f.at[pl.ds(e0 * 16, SUB * 16)], ae_v)
        pltpu.sync_copy(tc_hbm.at[src_i], ts_v)
        pltpu.sync_copy(tc_hbm.at[dst_i], td_v)

        def cmp(e, c2):
            ar = ae_v[pl.ds(e * 16, 16)]
            al = ts_v[e, 0:16] + td_v[e, 16:32] + ar
            al = jnp.where(al > 0, al, 0.2 * al)
            w = jnp.exp(al)
            w_v[pl.ds(e * 16, 16)] = w
            wa_v[e, 0:16] = w
            wa_v[e, 16:32] = ar
            return c2
        lax.fori_loop(0, SUB, cmp, 0)

        pltpu.sync_copy(wa_v, acc_sh.at[dst_i], add=True)
        pltpu.sync_copy(w_v, w16_out.at[pl.ds(e0 * 16, SUB * 16)])
        return carry

    lax.fori_loop(0, NSUB, sub_body, 0)
    plsc.subcore_barrier()
    @pl.when(sid == 0)
    def _writeout():
        pltpu.sync_copy(acc_sh, acc_out.at[cid])


def _s3_body(src1d, dst1d, x4r8, w16f, zrows128, agg_out,
             src_i, dst_i, w_v, xg_v, ov_v, acc_sh):
    cid = lax.axis_index("c")
    sid = lax.axis_index("s")
    wid = cid * 16 + sid

    for combo in range(8):
        hh = combo // 2

        @pl.when(sid == 0)
        def _init():
            pltpu.sync_copy(zrows128, acc_sh)
        plsc.subcore_barrier()

        def sub_body(s, carry):
            e0 = (wid * NSUB3 + s) * SUB3
            pltpu.sync_copy(src1d.at[pl.ds(e0, SUB3)], src_i)
            pltpu.sync_copy(dst1d.at[pl.ds(e0, SUB3)], dst_i)
            pltpu.sync_copy(w16f.at[pl.ds(e0 * 16, SUB3 * 16)], w_v)
            pltpu.sync_copy(x4r8.at[combo].at[src_i], xg_v)

            def cmp(e, c2):
                wreg = w_v[pl.ds(e * 16, 16)]
                ws = wreg[hh]
                for r in range(8):
                    sl = pl.ds(r * 16, 16)
                    ov_v[e, sl] = xg_v[e, sl] * ws
                return c2
            lax.fori_loop(0, SUB3, cmp, 0)

            pltpu.sync_copy(ov_v, acc_sh.at[dst_i], add=True)
            return carry

        lax.fori_loop(0, NSUB3, sub_body, 0)
        plsc.subcore_barrier()

        @pl.when(sid == 0)
        def _writeout():
            pltpu.sync_copy(acc_sh, agg_out.at[cid].at[combo])
        plsc.subcore_barrier()


def _run_s3(src_p, dst_p, x4r8, w16f, zrows128):
    f32 = jnp.float32
    mesh = plsc.VectorSubcoreMesh(core_axis_name="c", subcore_axis_name="s")
    fn = pl.kernel(
        _s3_body,
        mesh=mesh,
        out_type=[jax.ShapeDtypeStruct((2, 8, NP, 128), f32)],
        scratch_types=[
            pltpu.VMEM((SUB3,), jnp.int32),
            pltpu.VMEM((SUB3,), jnp.int32),
            pltpu.VMEM((SUB3 * 16,), f32),
            pltpu.VMEM((SUB3, 128), f32),
            pltpu.VMEM((SUB3, 128), f32),
            pltpu.VMEM_SHARED((NP, 128), f32),
        ],
    )
    return fn(src_p, dst_p, x4r8, w16f, zrows128)


def _run_s1(src_p, dst_p, tc_pad, ae16f, zrows128):
    f32 = jnp.float32
    mesh = plsc.VectorSubcoreMesh(core_axis_name="c", subcore_axis_name="s")
    fn = pl.kernel(
        _s1_body,
        mesh=mesh,
        out_type=[
            jax.ShapeDtypeStruct((EP * 16,), f32),
            jax.ShapeDtypeStruct((2, NP, 128), f32),
        ],
        scratch_types=[
            pltpu.VMEM((SUB,), jnp.int32),
            pltpu.VMEM((SUB,), jnp.int32),
            pltpu.VMEM((SUB, 128), f32),
            pltpu.VMEM((SUB, 128), f32),
            pltpu.VMEM((SUB * 16,), f32),
            pltpu.VMEM((SUB * 16,), f32),
            pltpu.VMEM((SUB, 128), f32),
            pltpu.VMEM_SHARED((NP, 128), f32),
        ],
    )
    return fn(src_p, dst_p, tc_pad, ae16f, zrows128)


def kernel(h, edge_attr, edge_index, W_src, att_src, att_dst, W_edge,
           att_edge, bias_gat, W_lin, b_lin, ln_gamma, ln_beta):
    f32 = jnp.float32
    Ws3 = W_src.reshape(D, H, D)
    Vs = (Ws3 * att_src).sum(-1)
    Vd = (Ws3 * att_dst).sum(-1)
    Ve = (W_edge.reshape(D, H, D) * att_edge).sum(-1)
    V128 = jnp.zeros((D, 128), f32)
    V128 = V128.at[:, 0:4].set(Vs).at[:, 16:20].set(Vd)
    Ve16 = jnp.pad(Ve, ((0, 0), (0, 12)))

    src, dst = edge_index[0], edge_index[1]

    x4r = pl.pallas_call(
        _k1a_body,
        grid=(H, 2, N // NB_N),
        in_specs=[
            pl.BlockSpec((NB_N, D), lambda hi, ci, ni: (ni, 0)),
            pl.BlockSpec((1, D, 128), lambda hi, ci, ni: (hi, 0, ci)),
        ],
        out_specs=pl.BlockSpec((1, 1, NB_N, 128),
                               lambda hi, ci, ni: (hi, ci, ni, 0)),
        out_shape=jax.ShapeDtypeStruct((H, 2, NP, 128), f32),
    )(h, Ws3.transpose(1, 0, 2))
    x4r8 = x4r.reshape(H * 2, NP, 128)

    Tc = pl.pallas_call(
        _k1b_body,
        grid=(N // NB_N,),
        in_specs=[
            pl.BlockSpec((NB_N, D), lambda ni: (ni, 0)),
            pl.BlockSpec((D, 128), lambda ni: (0, 0)),
        ],
        out_specs=pl.BlockSpec((NB_N, 128), lambda ni: (ni, 0)),
        out_shape=jax.ShapeDtypeStruct((N, 128), f32),
    )(h, V128)

    ea_p = jnp.pad(edge_attr, ((0, EP - E), (0, 0)))
    ae16p = pl.pallas_call(
        _k2_body,
        grid=(EP // NB_E,),
        in_specs=[
            pl.BlockSpec((NB_E, D), lambda ei: (ei, 0)),
            pl.BlockSpec((D, 16), lambda ei: (0, 0)),
        ],
        out_specs=pl.BlockSpec((NB_E, 16), lambda ei: (ei, 0)),
        out_shape=jax.ShapeDtypeStruct((EP, 16), f32),
    )(ea_p, Ve16)

    src_p = jnp.pad(src, (0, EP - E))
    dst_p = jnp.pad(dst, (0, EP - E), constant_values=N)
    tc_pad = jnp.pad(Tc, ((0, NP - N), (0, 0)))
    zrows128 = jnp.zeros((NP, 128), f32)
    w16f, parts = _run_s1(src_p, dst_p, tc_pad,
                          ae16p.reshape(EP * 16), zrows128)

    agg, = _run_s3(src_p, dst_p, x4r8, w16f, zrows128)

    out = pl.pallas_call(
        _k4_body,
        grid=(N // NB_N,),
        in_specs=[
            pl.BlockSpec((2, 8, NB_N, 128), lambda ni: (0, 0, ni, 0)),
            pl.BlockSpec((8, NB_N, 128), lambda ni: (0, ni, 0)),
            pl.BlockSpec((NB_N, 128), lambda ni: (ni, 0)),
            pl.BlockSpec((2, NB_N, 128), lambda ni: (0, ni, 0)),
            pl.BlockSpec((H * D, D), lambda ni: (0, 0)),
            pl.BlockSpec((1, H * D), lambda ni: (0, 0)),
            pl.BlockSpec((1, D), lambda ni: (0, 0)),
            pl.BlockSpec((1, D), lambda ni: (0, 0)),
            pl.BlockSpec((1, D), lambda ni: (0, 0)),
        ],
        out_specs=pl.BlockSpec((NB_N, D), lambda ni: (ni, 0)),
        out_shape=jax.ShapeDtypeStruct((N, D), f32),
    )(agg, x4r8, Tc, parts, W_lin,
      bias_gat.reshape(1, H * D), b_lin.reshape(1, D),
      ln_gamma.reshape(1, D), ln_beta.reshape(1, D))
    return out
